# baseline (speedup 1.0000x reference)
import functools

import jax
import jax.numpy as jnp
from jax import lax
from jax.experimental import pallas as pl
from jax.experimental.pallas import tpu as pltpu
from jax.experimental.pallas import tpu_sc as plsc

NC = 2
NS = 16
NW = NC * NS
D = 256
DP = 272


@functools.lru_cache(maxsize=None)
def _make_segsum(P, Ep, n_dst, n_acc, K):
    per_w = Ep // NW
    n_chunks = per_w // K
    assert per_w * NW == Ep and n_chunks * K == per_w and K % 8 == 0 and K <= 128
    assert n_chunks % 2 == 0 and n_acc >= n_dst + K and n_acc % 8 == 0
    n_pairs = n_chunks // 2
    rps = (-(-n_dst // NS) + 7) // 8 * 8
    n_full = n_dst // rps
    rem = n_dst - n_full * rps
    B0 = 32
    assert rps % B0 == 0 and rem % B0 == 0
    mesh = plsc.VectorSubcoreMesh(core_axis_name="c", subcore_axis_name="s",
                                  num_cores=NC, num_subcores=NS)

    @functools.partial(
        pl.kernel,
        out_type=jax.ShapeDtypeStruct((NC, n_dst, DP), jnp.float32),
        mesh=mesh,
        scratch_types=[
            pltpu.VMEM_SHARED((n_acc, DP), jnp.float32),
            pltpu.VMEM((n_chunks, K), jnp.int32),
            pltpu.VMEM((n_chunks, K), jnp.int32),
            pltpu.VMEM((2, K, DP), jnp.float32),
            pltpu.VMEM((B0, DP), jnp.float32),
            pltpu.SemaphoreType.DMA,
            pltpu.SemaphoreType.DMA,
        ],
        compiler_params=pltpu.CompilerParams(use_tc_tiling_on_sc=False),
    )
    def segsum(table_hbm, src_hbm, dst_hbm, zrow_hbm, parts_hbm,
               acc_sh, src_v, dst_v, rows_v, buf_v, sem0, sem1):
        c = lax.axis_index("c")
        s = lax.axis_index("s")
        wid = s * NC + c
        row_off = pl.multiple_of(s * rps, 8)

        pltpu.sync_copy(src_hbm.at[pl.ds(wid * n_chunks, n_chunks)], src_v)
        pltpu.sync_copy(dst_hbm.at[pl.ds(wid * n_chunks, n_chunks)], dst_v)

        pltpu.sync_copy(zrow_hbm, buf_v)

        @pl.when(s < n_full)
        def _zero_full():
            for j in range(rps // B0):
                pltpu.sync_copy(buf_v, acc_sh.at[pl.ds(row_off + j * B0, B0)])
        if rem:
            @pl.when(s == n_full)
            def _zero_rem():
                for j in range(rem // B0):
                    pltpu.sync_copy(
                        buf_v, acc_sh.at[pl.ds(n_full * rps + j * B0, B0)])
        plsc.subcore_barrier()

        sems = (sem0, sem1)

        def gather(ci, b):
            return pltpu.async_copy(table_hbm.at[src_v.at[ci]],
                                    rows_v.at[b], sems[b])

        def gwait(ci, b):
            pltpu.make_async_copy(table_hbm.at[src_v.at[ci]],
                                  rows_v.at[b], sems[b]).wait()

        def scatter(ci, b):
            pltpu.sync_copy(rows_v.at[b], acc_sh.at[dst_v.at[ci]], add=True)

        gather(0, 0)

        def pair(i, carry):
            c0 = i * 2
            gwait(c0, 0)
            gather(c0 + 1, 1)
            scatter(c0, 0)
            gwait(c0 + 1, 1)

            @pl.when(i < n_pairs - 1)
            def _next():
                gather(c0 + 2, 0)
            scatter(c0 + 1, 1)
            return carry

        lax.fori_loop(0, n_pairs, pair, 0)
        plsc.subcore_barrier()

        @pl.when(s < n_full)
        def _out_full():
            for j in range(rps // B0):
                pltpu.sync_copy(acc_sh.at[pl.ds(row_off + j * B0, B0)], buf_v)
                pltpu.sync_copy(buf_v,
                                parts_hbm.at[c, pl.ds(row_off + j * B0, B0)])
        if rem:
            @pl.when(s == n_full)
            def _out_rem():
                for j in range(rem // B0):
                    off_r = n_full * rps + j * B0
                    pltpu.sync_copy(acc_sh.at[pl.ds(off_r, B0)], buf_v)
                    pltpu.sync_copy(buf_v, parts_hbm.at[c, pl.ds(off_r, B0)])

    return segsum


def _t1r_body(x_ref, wr_ref, b_ref, r_ref):
    r_ref[...] = jnp.dot(x_ref[...], wr_ref[...],
                         preferred_element_type=jnp.float32) + b_ref[...]


def _t1r(x4k, W_r, b):
    M, H = 4000, 1024
    BM = 800
    return pl.pallas_call(
        _t1r_body,
        grid=(M // BM,),
        in_specs=[
            pl.BlockSpec((BM, D), lambda i: (i, 0)),
            pl.BlockSpec((D, H), lambda i: (0, 0)),
            pl.BlockSpec((1, H), lambda i: (0, 0)),
        ],
        out_specs=pl.BlockSpec((BM, H), lambda i: (i, 0)),
        out_shape=jax.ShapeDtypeStruct((M, H), jnp.float32),
    )(x4k, W_r, b.reshape(1, H))


def _t1f_body(parts_ref, wl_ref, r_ref, h_ref):
    s = parts_ref[0] + parts_ref[1]
    mean = s[:, :D] / jnp.maximum(s[:, D:D + 1], 1.0)
    acc = jnp.dot(mean, wl_ref[...], preferred_element_type=jnp.float32)
    h_ref[...] = jnp.maximum(acc + r_ref[...], 0.0)


def _t1f(parts, W_l, r1):
    M, H = 4000, 1024
    BM = 800
    return pl.pallas_call(
        _t1f_body,
        grid=(M // BM,),
        in_specs=[
            pl.BlockSpec((NC, BM, DP), lambda i: (0, i, 0)),
            pl.BlockSpec((D, H), lambda i: (0, 0)),
            pl.BlockSpec((BM, H), lambda i: (i, 0)),
        ],
        out_specs=pl.BlockSpec((BM, H), lambda i: (i, 0)),
        out_shape=jax.ShapeDtypeStruct((M, H), jnp.float32),
    )(parts, W_l, r1)


def _t2z_body(h_ref, wlp_ref, zp_ref):
    z = jnp.dot(h_ref[...], wlp_ref[...], preferred_element_type=jnp.float32)
    col = lax.broadcasted_iota(jnp.int32, z.shape, 1)
    zp_ref[...] = z + jnp.where(col == D, 1.0, 0.0)


def _t2z(h1k, wlp):
    M, H = 1024, 1024
    BM = 512
    return pl.pallas_call(
        _t2z_body,
        grid=(M // BM,),
        in_specs=[
            pl.BlockSpec((BM, H), lambda i: (i, 0)),
            pl.BlockSpec((H, DP), lambda i: (0, 0)),
        ],
        out_specs=pl.BlockSpec((BM, DP), lambda i: (i, 0)),
        out_shape=jax.ShapeDtypeStruct((M, DP), jnp.float32),
    )(h1k, wlp)


def _t2r_body(h_ref, wr_ref, r2_ref):
    r2_ref[...] = jnp.dot(h_ref[...], wr_ref[...],
                          preferred_element_type=jnp.float32)


def _t2r(h1k, W_r):
    M, H = 1024, 1024
    BM = 512
    return pl.pallas_call(
        _t2r_body,
        grid=(M // BM,),
        in_specs=[
            pl.BlockSpec((BM, H), lambda i: (i, 0)),
            pl.BlockSpec((H, D), lambda i: (0, 0)),
        ],
        out_specs=pl.BlockSpec((BM, D), lambda i: (i, 0)),
        out_shape=jax.ShapeDtypeStruct((M, D), jnp.float32),
    )(h1k, W_r)


def _t3_body(parts_ref, r2_ref, b_ref, out_ref):
    s = parts_ref[0] + parts_ref[1]
    mean = s[:, :D] / jnp.maximum(s[:, D:D + 1], 1.0)
    o = mean + r2_ref[...] + b_ref[...]
    m = jnp.max(o, axis=1, keepdims=True)
    e = jnp.exp(o - m)
    lse = jnp.log(jnp.sum(e, axis=1, keepdims=True))
    out_ref[...] = (o - m) - lse


def _t3(parts, r2, b):
    M = 1024
    return pl.pallas_call(
        _t3_body,
        grid=(1,),
        in_specs=[
            pl.BlockSpec((NC, M, DP), lambda i: (0, 0, 0)),
            pl.BlockSpec((M, D), lambda i: (0, 0)),
            pl.BlockSpec((1, D), lambda i: (0, 0)),
        ],
        out_specs=pl.BlockSpec((M, D), lambda i: (0, 0)),
        out_shape=jax.ShapeDtypeStruct((M, D), jnp.float32),
    )(parts, r2, b.reshape(1, D))


def _pad_edges(edge_index, E, Ep, n_dst):
    K = 64
    pad = Ep - E
    spread = jnp.arange(pad, dtype=jnp.int32) % K
    src = jnp.concatenate([edge_index[0], spread])
    dst = jnp.concatenate([edge_index[1], n_dst + spread])
    return src.reshape(Ep // K, K), dst.reshape(Ep // K, K)


def kernel(x, edge_index1, edge_index2, W_l1, W_r1, b1, W_l2, W_r2, b2,
           n_dst1, n_dst2):
    off1 = n_dst1 - 4000
    x4k = lax.dynamic_slice_in_dim(x, off1, 4000, axis=0)
    xplus = jnp.concatenate(
        [x4k, jnp.ones((4000, 1), jnp.float32), jnp.zeros((4000, 15), jnp.float32)],
        axis=1)
    z1 = jnp.zeros((32, DP), jnp.float32)
    src1, dst1 = _pad_edges(edge_index1, 160000, 163840, 4000)
    parts1 = _make_segsum(P=4000, Ep=163840, n_dst=4000, n_acc=4096, K=64)(
        xplus, src1, dst1, z1)
    r1 = _t1r(x4k, W_r1, b1)
    h = _t1f(parts1, W_l1, r1)

    off2 = n_dst2 - 1024
    h1k = lax.dynamic_slice_in_dim(h, off2, 1024, axis=0)
    wlp = jnp.pad(W_l2, ((0, 0), (0, DP - D)))
    zp = _t2z(h1k, wlp)
    r2 = _t2r(h1k, W_r2)
    z2 = jnp.zeros((32, DP), jnp.float32)
    src2, dst2 = _pad_edges(edge_index2, 64000, 65536, 1024)
    parts2 = _make_segsum(P=1024, Ep=65536, n_dst=1024, n_acc=1152, K=64)(
        zp, src2, dst2, z2)
    return _t3(parts2, r2, b2)


# speedup vs baseline: 1.6520x; 1.6520x over previous
import functools

import jax
import jax.numpy as jnp
from jax import lax
from jax.experimental import pallas as pl
from jax.experimental.pallas import tpu as pltpu
from jax.experimental.pallas import tpu_sc as plsc

NC = 2
NS = 16
NW = NC * NS
D = 256
CW = 16
K = 64


@functools.lru_cache(maxsize=None)
def _make_segsum(P, Ep, n_keep, n_acc):
    per_w = Ep // NW
    assert per_w * NW == Ep and per_w % K == 0 and per_w % 16 == 0
    ncap = per_w + 2 * K
    assert n_acc >= n_keep + 16 and n_acc % 8 == 0
    rps = (-(-n_keep // NS) + 7) // 8 * 8
    n_full = n_keep // rps
    rem = n_keep - n_full * rps
    zps = (-(-n_acc // NS) + 7) // 8 * 8
    z_full = n_acc // zps
    z_rem = n_acc - z_full * zps
    B0 = 16
    assert rps % B0 == 0 and rem % B0 == 0 and zps % B0 == 0 and z_rem % B0 == 0
    mesh = plsc.VectorSubcoreMesh(core_axis_name="c", subcore_axis_name="s",
                                  num_cores=NC, num_subcores=NS)

    @functools.partial(
        pl.kernel,
        out_type=(jax.ShapeDtypeStruct((NC, n_keep, D), jnp.float32),
                  jax.ShapeDtypeStruct((NC, n_keep, CW), jnp.float32)),
        mesh=mesh,
        scratch_types=[
            pltpu.VMEM_SHARED((n_acc, D), jnp.float32),
            pltpu.VMEM_SHARED((n_acc, CW), jnp.float32),
            pltpu.VMEM((per_w,), jnp.int32),
            pltpu.VMEM((per_w,), jnp.int32),
            pltpu.VMEM((ncap,), jnp.int32),
            pltpu.VMEM((ncap,), jnp.int32),
            pltpu.VMEM((2, K, D), jnp.float32),
            pltpu.VMEM((K, CW), jnp.float32),
            pltpu.VMEM((B0, D), jnp.float32),
            pltpu.VMEM((B0, CW), jnp.float32),
            pltpu.SemaphoreType.DMA,
            pltpu.SemaphoreType.DMA,
        ],
        compiler_params=pltpu.CompilerParams(use_tc_tiling_on_sc=False,
                                             needs_layout_passes=False),
    )
    def segsum(table_hbm, src_hbm, dst_hbm, zrow_hbm, zcnt_hbm, onesrow_hbm,
               parts_hbm, cparts_hbm,
               acc_sh, cacc_sh, src_v, dst_v, srcc_v, dstc_v, rows_v, ones_v,
               buf_v, bufc_v, sem0, sem1):
        c = lax.axis_index("c")
        s = lax.axis_index("s")
        wid = s * NC + c

        pltpu.sync_copy(src_hbm.at[pl.ds(wid * per_w, per_w)], src_v)
        pltpu.sync_copy(dst_hbm.at[pl.ds(wid * per_w, per_w)], dst_v)
        pltpu.sync_copy(zrow_hbm, buf_v)
        pltpu.sync_copy(zcnt_hbm, bufc_v)
        pltpu.sync_copy(onesrow_hbm, ones_v)

        zoff = pl.multiple_of(s * zps, 8)

        @pl.when(s < z_full)
        def _zero_full():
            for j in range(zps // B0):
                pltpu.sync_copy(buf_v, acc_sh.at[pl.ds(zoff + j * B0, B0)])
            for j in range(zps // B0):
                pltpu.sync_copy(bufc_v, cacc_sh.at[pl.ds(zoff + j * B0, B0)])
        if z_rem:
            @pl.when(s == z_full)
            def _zero_rem():
                for j in range(z_rem // B0):
                    pltpu.sync_copy(
                        buf_v, acc_sh.at[pl.ds(z_full * zps + j * B0, B0)])
                for j in range(z_rem // B0):
                    pltpu.sync_copy(
                        bufc_v, cacc_sh.at[pl.ds(z_full * zps + j * B0, B0)])

        lane = lax.iota(jnp.int32, 16)

        def fill(g, carry):
            srcc_v[pl.ds(g * 16, 16)] = lane
            dstc_v[pl.ds(g * 16, 16)] = lane + n_keep
            return carry

        lax.fori_loop(0, ncap // 16, fill, 0)

        def compact(g, off):
            sv = src_v[pl.ds(g * 16, 16)]
            dv = dst_v[pl.ds(g * 16, 16)]
            m = dv < n_keep
            packed = jnp.bitwise_or(jnp.left_shift(sv, 12), dv)
            _, pv = plsc.sort_key_val(dv, packed)
            nk = plsc.all_reduce_population_count(m)
            keep = lane < nk
            pos = off + lane
            plsc.store_scatter(srcc_v, [pos], jnp.right_shift(pv, 12), mask=keep)
            plsc.store_scatter(dstc_v, [pos], jnp.bitwise_and(pv, 4095), mask=keep)
            return off + nk

        offv = lax.fori_loop(0, per_w // 16, compact,
                             jnp.zeros((16,), jnp.int32))
        n_edges = jnp.max(offv)
        n_pairs = (n_edges + 2 * K - 1) // (2 * K)

        plsc.subcore_barrier()

        sems = (sem0, sem1)

        def gather(ci, b):
            pltpu.async_copy(table_hbm.at[srcc_v.at[pl.ds(ci * K, K)]],
                             rows_v.at[b], sems[b])

        def gwait(ci, b):
            pltpu.make_async_copy(table_hbm.at[srcc_v.at[pl.ds(ci * K, K)]],
                                  rows_v.at[b], sems[b]).wait()

        def scatter(ci, b):
            idx = dstc_v.at[pl.ds(ci * K, K)]
            pltpu.sync_copy(rows_v.at[b], acc_sh.at[idx], add=True)
            pltpu.sync_copy(ones_v, cacc_sh.at[idx], add=True)

        @pl.when(n_pairs > 0)
        def _prologue():
            gather(0, 0)

        def pair(i, carry):
            c0 = i * 2
            gwait(c0, 0)
            gather(c0 + 1, 1)
            scatter(c0, 0)
            gwait(c0 + 1, 1)

            @pl.when(i < n_pairs - 1)
            def _next():
                gather(c0 + 2, 0)
            scatter(c0 + 1, 1)
            return carry

        lax.fori_loop(0, n_pairs, pair, 0)
        plsc.subcore_barrier()

        row_off = pl.multiple_of(s * rps, 8)

        @pl.when(s < n_full)
        def _out_full():
            for j in range(rps // B0):
                pltpu.sync_copy(acc_sh.at[pl.ds(row_off + j * B0, B0)], buf_v)
                pltpu.sync_copy(buf_v,
                                parts_hbm.at[c, pl.ds(row_off + j * B0, B0)])
            for j in range(rps // B0):
                pltpu.sync_copy(cacc_sh.at[pl.ds(row_off + j * B0, B0)], bufc_v)
                pltpu.sync_copy(bufc_v,
                                cparts_hbm.at[c, pl.ds(row_off + j * B0, B0)])
        if rem:
            @pl.when(s == n_full)
            def _out_rem():
                for j in range(rem // B0):
                    off_r = n_full * rps + j * B0
                    pltpu.sync_copy(acc_sh.at[pl.ds(off_r, B0)], buf_v)
                    pltpu.sync_copy(buf_v, parts_hbm.at[c, pl.ds(off_r, B0)])
                for j in range(rem // B0):
                    off_r = n_full * rps + j * B0
                    pltpu.sync_copy(cacc_sh.at[pl.ds(off_r, B0)], bufc_v)
                    pltpu.sync_copy(bufc_v, cparts_hbm.at[c, pl.ds(off_r, B0)])

    return segsum


def _pad_edges(edge_index, E, Ep, dump):
    pad = Ep - E
    spread = jnp.arange(pad, dtype=jnp.int32) % K
    src = jnp.concatenate([edge_index[0], spread])
    dst = jnp.concatenate([edge_index[1], dump + spread])
    return src, dst


def _mean(parts_ref, cnt_ref):
    s = parts_ref[0] + parts_ref[1]
    cnt = cnt_ref[0, :, :1] + cnt_ref[1, :, :1]
    return s / jnp.maximum(cnt, 1.0)


def _t1_body(parts_ref, cnt_ref, x_ref, wl_ref, wr_ref, b_ref, h_ref):
    acc = jnp.dot(_mean(parts_ref, cnt_ref), wl_ref[...],
                  preferred_element_type=jnp.float32)
    acc = acc + jnp.dot(x_ref[...], wr_ref[...],
                        preferred_element_type=jnp.float32)
    h_ref[...] = jnp.maximum(acc + b_ref[...], 0.0)


def _t1(parts, cnt, x1k, W_l, W_r, b):
    M, H = 1024, 1024
    BM = 512
    return pl.pallas_call(
        _t1_body,
        grid=(M // BM,),
        in_specs=[
            pl.BlockSpec((NC, BM, D), lambda i: (0, i, 0)),
            pl.BlockSpec((NC, BM, CW), lambda i: (0, i, 0)),
            pl.BlockSpec((BM, D), lambda i: (i, 0)),
            pl.BlockSpec((D, H), lambda i: (0, 0)),
            pl.BlockSpec((D, H), lambda i: (0, 0)),
            pl.BlockSpec((1, H), lambda i: (0, 0)),
        ],
        out_specs=pl.BlockSpec((BM, H), lambda i: (i, 0)),
        out_shape=jax.ShapeDtypeStruct((M, H), jnp.float32),
    )(parts, cnt, x1k, W_l, W_r, b.reshape(1, H))


def _t2_body(h_ref, wl_ref, wr_ref, z_ref, r2_ref):
    hblk = h_ref[...]
    z_ref[...] = jnp.dot(hblk, wl_ref[...], preferred_element_type=jnp.float32)
    r2_ref[...] = jnp.dot(hblk, wr_ref[...], preferred_element_type=jnp.float32)


def _t2(h1k, W_l, W_r):
    M, H = 1024, 1024
    BM = 512
    return pl.pallas_call(
        _t2_body,
        grid=(M // BM,),
        in_specs=[
            pl.BlockSpec((BM, H), lambda i: (i, 0)),
            pl.BlockSpec((H, D), lambda i: (0, 0)),
            pl.BlockSpec((H, D), lambda i: (0, 0)),
        ],
        out_specs=[
            pl.BlockSpec((BM, D), lambda i: (i, 0)),
            pl.BlockSpec((BM, D), lambda i: (i, 0)),
        ],
        out_shape=[
            jax.ShapeDtypeStruct((M, D), jnp.float32),
            jax.ShapeDtypeStruct((M, D), jnp.float32),
        ],
    )(h1k, W_l, W_r)


def _t3_body(parts_ref, cnt_ref, r2_ref, b_ref, out_ref):
    o = _mean(parts_ref, cnt_ref) + r2_ref[...] + b_ref[...]
    m = jnp.max(o, axis=1, keepdims=True)
    e = jnp.exp(o - m)
    lse = jnp.log(jnp.sum(e, axis=1, keepdims=True))
    out_ref[...] = (o - m) - lse


def _t3(parts, cnt, r2, b):
    M = 1024
    return pl.pallas_call(
        _t3_body,
        grid=(1,),
        in_specs=[
            pl.BlockSpec((NC, M, D), lambda i: (0, 0, 0)),
            pl.BlockSpec((NC, M, CW), lambda i: (0, 0, 0)),
            pl.BlockSpec((M, D), lambda i: (0, 0)),
            pl.BlockSpec((1, D), lambda i: (0, 0)),
        ],
        out_specs=pl.BlockSpec((M, D), lambda i: (0, 0)),
        out_shape=jax.ShapeDtypeStruct((M, D), jnp.float32),
    )(parts, cnt, r2, b.reshape(1, D))


def kernel(x, edge_index1, edge_index2, W_l1, W_r1, b1, W_l2, W_r2, b2,
           n_dst1, n_dst2):
    off1 = n_dst1 - 4000
    x4k = lax.dynamic_slice_in_dim(x, off1, 4000, axis=0)
    x1k = x4k[:1024]
    zrow = jnp.zeros((16, D), jnp.float32)
    zcnt = jnp.zeros((16, CW), jnp.float32)
    onesrow = jnp.ones((K, CW), jnp.float32)

    src1, dst1 = _pad_edges(edge_index1, 160000, 163840, 4000)
    parts1, cnt1 = _make_segsum(P=4000, Ep=163840, n_keep=1024, n_acc=4096)(
        x4k, src1, dst1, zrow, zcnt, onesrow)
    h1k = _t1(parts1, cnt1, x1k, W_l1, W_r1, b1)

    z, r2 = _t2(h1k, W_l2, W_r2)
    src2, dst2 = _pad_edges(edge_index2, 64000, 65536, 1024)
    parts2, cnt2 = _make_segsum(P=1024, Ep=65536, n_keep=1024, n_acc=2048)(
        z, src2, dst2, zrow, zcnt, onesrow)
    return _t3(parts2, cnt2, r2, b2)
